# baseline (speedup 1.0000x reference)
import jax
import jax.numpy as jnp
from jax import lax
from jax.experimental import pallas as pl
from jax.experimental.pallas import tpu as pltpu
from jax.experimental.pallas import tpu_sc as plsc

B = 16384
F = 26
CARD = 100000
NC = 2
NS = 16
NW = NC * NS
B_PER_W = B // NW
N_PER_W = B_PER_W * F
L = 16
GROUPS = B_PER_W // L


def _body(xb_hbm, tbl_hbm, bias_hbm, out_hbm, x_v, idx_v, vals_v, out_v,
          bias_v, sem):
    wid = lax.axis_index("c") * NS + lax.axis_index("s")

    pltpu.sync_copy(xb_hbm.at[wid], x_v)
    pltpu.sync_copy(bias_hbm, bias_v)

    def build(j, _):
        f = j >> 5
        idx_v[pl.ds(j * L, L)] = x_v[pl.ds(j * L, L)] + f * CARD
        return 0

    lax.fori_loop(0, F * GROUPS, build, 0)

    pltpu.async_copy(tbl_hbm.at[idx_v], vals_v, sem).wait()

    bias16 = bias_v[...]

    def reduce(s, _):
        acc = bias16
        for f in range(F):
            acc = acc + vals_v[pl.ds(f * B_PER_W + s * L, L)]
        out_v[pl.ds(s * L, L)] = acc
        return 0

    lax.fori_loop(0, GROUPS, reduce, 0)

    pltpu.sync_copy(out_v, out_hbm.at[pl.ds(wid * B_PER_W, B_PER_W)])


@jax.jit
def _run(xb, tbl_flat, bias16):
    mesh = plsc.VectorSubcoreMesh(core_axis_name="c", subcore_axis_name="s")
    return pl.kernel(
        _body,
        out_type=jax.ShapeDtypeStruct((B,), jnp.float32),
        mesh=mesh,
        scratch_types=[
            pltpu.VMEM((N_PER_W,), jnp.int32),
            pltpu.VMEM((N_PER_W,), jnp.int32),
            pltpu.VMEM((N_PER_W,), jnp.float32),
            pltpu.VMEM((B_PER_W,), jnp.float32),
            pltpu.VMEM((L,), jnp.float32),
            pltpu.SemaphoreType.DMA,
        ],
    )(xb, tbl_flat, bias16)


def kernel(x, table, bias):
    xb = (x.astype(jnp.int32)
          .reshape(NW, B_PER_W, F)
          .transpose(0, 2, 1)
          .reshape(NW, N_PER_W))
    tbl_flat = table.reshape(-1)
    bias16 = jnp.broadcast_to(bias.astype(jnp.float32), (L,))
    out = _run(xb, tbl_flat, bias16)
    return out.reshape(B, 1)


# speedup vs baseline: 1.0461x; 1.0461x over previous
import jax
import jax.numpy as jnp
from jax import lax
from jax.experimental import pallas as pl
from jax.experimental.pallas import tpu as pltpu
from jax.experimental.pallas import tpu_sc as plsc

B = 16384
F = 26
CARD = 100000
NC = 2
NS = 16
NW = NC * NS
B_PER_W = B // NW
N_PER_W = B_PER_W * F
L = 16
GROUPS = B_PER_W // L


def _body(xt_hbm, tbl_hbm, bias_hbm, out_hbm, x_v, idx_v, vals_v, out_v,
          bias_v, sem):
    wid = lax.axis_index("c") * NS + lax.axis_index("s")
    sbase = wid * B_PER_W

    pltpu.sync_copy(xt_hbm.at[:, pl.ds(sbase, B_PER_W)], x_v)
    pltpu.sync_copy(bias_hbm, bias_v)

    def build(j, _):
        f = j >> 5
        r = j & (GROUPS - 1)
        idx_v[pl.ds(j * L, L)] = x_v[f, pl.ds(r * L, L)] + f * CARD
        return 0

    lax.fori_loop(0, F * GROUPS, build, 0)

    pltpu.async_copy(tbl_hbm.at[idx_v], vals_v, sem).wait()

    bias16 = bias_v[...]

    def reduce(s, _):
        acc = bias16
        for f in range(F):
            acc = acc + vals_v[pl.ds(f * B_PER_W + s * L, L)]
        out_v[pl.ds(s * L, L)] = acc
        return 0

    lax.fori_loop(0, GROUPS, reduce, 0)

    pltpu.sync_copy(out_v, out_hbm.at[pl.ds(sbase, B_PER_W)])


@jax.jit
def _run(xt, tbl_flat, bias16):
    mesh = plsc.VectorSubcoreMesh(core_axis_name="c", subcore_axis_name="s")
    return pl.kernel(
        _body,
        out_type=jax.ShapeDtypeStruct((B,), jnp.float32),
        mesh=mesh,
        scratch_types=[
            pltpu.VMEM((F, B_PER_W), jnp.int32),
            pltpu.VMEM((N_PER_W,), jnp.int32),
            pltpu.VMEM((N_PER_W,), jnp.float32),
            pltpu.VMEM((B_PER_W,), jnp.float32),
            pltpu.VMEM((L,), jnp.float32),
            pltpu.SemaphoreType.DMA,
        ],
    )(xt, tbl_flat, bias16)


def kernel(x, table, bias):
    xt = x.astype(jnp.int32).T
    tbl_flat = jnp.concatenate([table[k * 650000:(k + 1) * 650000, 0] for k in range(4)])
    bias16 = jnp.broadcast_to(bias.astype(jnp.float32), (L,))
    out = _run(xt, tbl_flat, bias16)
    return out.reshape(B, 1)


# speedup vs baseline: 1.3217x; 1.2635x over previous
import jax
import jax.numpy as jnp
from jax import lax
from jax.experimental import pallas as pl
from jax.experimental.pallas import tpu as pltpu
from jax.experimental.pallas import tpu_sc as plsc

B = 16384
F = 26
CARD = 100000
NC = 2
NS = 16
NW = NC * NS
B_PER_W = B // NW
N_PER_W = B_PER_W * F
L = 16
GROUPS = B_PER_W // L
TROWS = CARD * F


def _body(xb_hbm, tbl_hbm, bias_hbm, out_hbm, x_v, idx_v, vals_v, out_v,
          bias_v, sem):
    wid = lax.axis_index("c") * NS + lax.axis_index("s")

    pltpu.sync_copy(xb_hbm.at[wid], x_v)
    pltpu.sync_copy(bias_hbm, bias_v)

    def build(j, _):
        f = j >> 5
        idx_v[pl.ds(j * L, L)] = x_v[pl.ds(j * L, L)] + f * CARD
        return 0

    lax.fori_loop(0, F * GROUPS, build, 0)

    tbl_flat = tbl_hbm.reshape(1, TROWS).at[0]
    pltpu.async_copy(tbl_flat.at[idx_v], vals_v, sem).wait()

    bias16 = bias_v[...]

    def reduce(s, _):
        acc = bias16
        for f in range(F):
            acc = acc + vals_v[pl.ds(f * B_PER_W + s * L, L)]
        out_v[pl.ds(s * L, L)] = acc
        return 0

    lax.fori_loop(0, GROUPS, reduce, 0)

    pltpu.sync_copy(out_v, out_hbm.at[pl.ds(wid * B_PER_W, B_PER_W)])


@jax.jit
def _run(xb, tbl, bias16):
    mesh = plsc.VectorSubcoreMesh(core_axis_name="c", subcore_axis_name="s")
    return pl.kernel(
        _body,
        out_type=jax.ShapeDtypeStruct((B,), jnp.float32),
        mesh=mesh,
        scratch_types=[
            pltpu.VMEM((N_PER_W,), jnp.int32),
            pltpu.VMEM((N_PER_W,), jnp.int32),
            pltpu.VMEM((N_PER_W,), jnp.float32),
            pltpu.VMEM((B_PER_W,), jnp.float32),
            pltpu.VMEM((L,), jnp.float32),
            pltpu.SemaphoreType.DMA,
        ],
    )(xb, tbl, bias16)


def kernel(x, table, bias):
    xb = (x.astype(jnp.int32)
          .reshape(NW, B_PER_W, F)
          .transpose(0, 2, 1)
          .reshape(NW, N_PER_W))
    bias16 = jnp.broadcast_to(bias.astype(jnp.float32), (L,))
    out = _run(xb, table, bias16)
    return out.reshape(B, 1)


# speedup vs baseline: 1.4292x; 1.0813x over previous
import jax
import jax.numpy as jnp
from jax import lax
from jax.experimental import pallas as pl
from jax.experimental.pallas import tpu as pltpu
from jax.experimental.pallas import tpu_sc as plsc

B = 16384
F = 26
CARD = 100000
NC = 2
NS = 16
NW = NC * NS
B_PER_W = B // NW
N_PER_W = B_PER_W * F
L = 16
GROUPS = B_PER_W // L


def _body(xt_hbm, tbl_hbm, bias_hbm, out_hbm, x_v, idx_v, vals_v, out_v,
          bias_v, sem):
    wid = lax.axis_index("c") * NS + lax.axis_index("s")
    sbase = wid * B_PER_W

    pltpu.sync_copy(xt_hbm.at[:, pl.ds(sbase, B_PER_W)], x_v)
    pltpu.sync_copy(bias_hbm, bias_v)

    def build(j, _):
        f = j >> 5
        r = j & (GROUPS - 1)
        idx_v[pl.ds(j * L, L)] = x_v[f, pl.ds(r * L, L)] + f * CARD
        return 0

    lax.fori_loop(0, F * GROUPS, build, 0)

    pltpu.async_copy(tbl_hbm.at[idx_v], vals_v, sem).wait()

    bias16 = bias_v[...]

    def reduce(s, _):
        acc = bias16
        for f in range(F):
            acc = acc + vals_v[pl.ds(f * B_PER_W + s * L, L)]
        out_v[pl.ds(s * L, L)] = acc
        return 0

    lax.fori_loop(0, GROUPS, reduce, 0)

    pltpu.sync_copy(out_v, out_hbm.at[pl.ds(sbase, B_PER_W)])


@jax.jit
def _run(xt, tbl_flat, bias16):
    mesh = plsc.VectorSubcoreMesh(core_axis_name="c", subcore_axis_name="s")
    return pl.kernel(
        _body,
        out_type=jax.ShapeDtypeStruct((B,), jnp.float32),
        mesh=mesh,
        scratch_types=[
            pltpu.VMEM((F, B_PER_W), jnp.int32),
            pltpu.VMEM((N_PER_W,), jnp.int32),
            pltpu.VMEM((N_PER_W,), jnp.float32),
            pltpu.VMEM((B_PER_W,), jnp.float32),
            pltpu.VMEM((L,), jnp.float32),
            pltpu.SemaphoreType.DMA,
        ],
    )(xt, tbl_flat, bias16)


def kernel(x, table, bias):
    xt = x.astype(jnp.int32).T
    tbl_flat = jnp.concatenate([table[:1300000, 0], table[1300000:, 0]])
    bias16 = jnp.broadcast_to(bias.astype(jnp.float32), (L,))
    out = _run(xt, tbl_flat, bias16)
    return out.reshape(B, 1)
